# baseline (speedup 1.0000x reference)
import jax
import jax.numpy as jnp
from jax.experimental import pallas as pl

_DEPTH = 1000
_LANES = 128
_PERIOD = 125
_S = 4000


def _patterns():
    f = jnp.arange(_PERIOD * _LANES, dtype=jnp.int32).reshape(_PERIOD, _LANES)
    d = f % _DEPTH
    p = f // _DEPTH
    p0 = (jnp.arange(_PERIOD, dtype=jnp.int32) * _LANES) // _DEPTH
    in_a = p == p0[:, None]
    dpa = jnp.where(in_a, d, -1)
    dpb = jnp.where(~in_a, d, -1)
    return dpa, dpb


def _onehot_block(dpa_ref, dpb_ref, ia_ref, ib_ref, out_ref):
    reps = _S // _PERIOD
    dpa = jnp.tile(dpa_ref[...], (reps, 1))
    dpb = jnp.tile(dpb_ref[...], (reps, 1))
    ia = ia_ref[...]
    ib = ib_ref[...]
    mask = (dpa == ia) | (dpb == ib)
    out_ref[...] = jnp.where(mask, jnp.float32(1.0), jnp.float32(0.0))


def kernel(indices):
    B, P = indices.shape
    n_rows = B * P
    n_flat = n_rows * _DEPTH // _LANES
    idx_flat = indices.astype(jnp.int32).reshape(-1)
    r = jnp.arange(n_flat, dtype=jnp.int32)
    p0 = (r * _LANES) // _DEPTH
    idx_a = idx_flat[p0].reshape(n_flat, 1)
    idx_b = idx_flat[jnp.minimum(p0 + 1, n_rows - 1)].reshape(n_flat, 1)
    dpa, dpb = _patterns()

    out2 = pl.pallas_call(
        _onehot_block,
        grid=(n_flat // _S,),
        in_specs=[
            pl.BlockSpec((_PERIOD, _LANES), lambda i: (0, 0)),
            pl.BlockSpec((_PERIOD, _LANES), lambda i: (0, 0)),
            pl.BlockSpec((_S, 1), lambda i: (i, 0)),
            pl.BlockSpec((_S, 1), lambda i: (i, 0)),
        ],
        out_specs=pl.BlockSpec((_S, _LANES), lambda i: (i, 0)),
        out_shape=jax.ShapeDtypeStruct((n_flat, _LANES), jnp.float32),
    )(dpa, dpb, idx_a, idx_b)
    return out2.reshape(B, P, _DEPTH)


# speedup vs baseline: 7.2797x; 7.2797x over previous
import jax
import jax.numpy as jnp
from jax.experimental import pallas as pl

_DEPTH = 1000
_LANES = 128
_PERIOD = 125
_S = 4000


def _patterns():
    f = jnp.arange(_PERIOD * _LANES, dtype=jnp.int32).reshape(_PERIOD, _LANES)
    d = f % _DEPTH
    p = f // _DEPTH
    p0 = (jnp.arange(_PERIOD, dtype=jnp.int32) * _LANES) // _DEPTH
    in_a = p == p0[:, None]
    dpa = jnp.where(in_a, d, -1)
    dpb = jnp.where(~in_a, d, -1)
    return dpa, dpb


def _onehot_block(dpa_ref, dpb_ref, ia_ref, ib_ref, out_ref):
    reps = _S // _PERIOD
    dpa = jnp.tile(dpa_ref[...], (reps, 1))
    dpb = jnp.tile(dpb_ref[...], (reps, 1))
    ia = ia_ref[...]
    ib = ib_ref[...]
    mask = (dpa == ia) | (dpb == ib)
    out_ref[...] = jnp.where(mask, jnp.float32(1.0), jnp.float32(0.0))


def _row_select_mats():
    t = (jnp.arange(_PERIOD) * _LANES) // _DEPTH
    g = _PERIOD * _LANES // _DEPTH
    ma = (t[None, :] == jnp.arange(g)[:, None]).astype(jnp.float32)
    tb = jnp.minimum(t + 1, g - 1)
    mb = (tb[None, :] == jnp.arange(g)[:, None]).astype(jnp.float32)
    return ma, mb


def kernel(indices):
    B, P = indices.shape
    n_rows = B * P
    n_flat = n_rows * _DEPTH // _LANES
    g = _PERIOD * _LANES // _DEPTH
    idx_g = indices.astype(jnp.float32).reshape(-1, g)
    ma, mb = _row_select_mats()
    idx_a = (idx_g @ ma).astype(jnp.int32).reshape(n_flat, 1)
    idx_b = (idx_g @ mb).astype(jnp.int32).reshape(n_flat, 1)
    dpa, dpb = _patterns()

    out2 = pl.pallas_call(
        _onehot_block,
        grid=(n_flat // _S,),
        in_specs=[
            pl.BlockSpec((_PERIOD, _LANES), lambda i: (0, 0)),
            pl.BlockSpec((_PERIOD, _LANES), lambda i: (0, 0)),
            pl.BlockSpec((_S, 1), lambda i: (i, 0)),
            pl.BlockSpec((_S, 1), lambda i: (i, 0)),
        ],
        out_specs=pl.BlockSpec((_S, _LANES), lambda i: (i, 0)),
        out_shape=jax.ShapeDtypeStruct((n_flat, _LANES), jnp.float32),
    )(dpa, dpb, idx_a, idx_b)
    return out2.reshape(B, P, _DEPTH)
